# baseline (speedup 1.0000x reference)
import functools

import jax
import jax.numpy as jnp
from jax import lax
from jax.experimental import pallas as pl
from jax.experimental.pallas import tpu as pltpu
from jax.experimental.pallas import tpu_sc as plsc

B = 16384
LSEQ = 200
VOCAB = 1000
D = 10
DP = 5
NC, NS = 2, 16
NW = NC * NS
RPW = B // NW
RB = 16
CH = 128
NCH = RPW // CH
NBC = CH // RB


def _sc_body(x_hbm, emb_hbm, wb_hbm, out_hbm, x_v0, x_v1, emb_v, wb_v, out_v,
             sem0, sem1):
    wid = lax.axis_index("s") * NC + lax.axis_index("c")
    base = wid * RPW
    pltpu.sync_copy(emb_hbm, emb_v)
    pltpu.sync_copy(wb_hbm, wb_v)

    lanes = lax.iota(jnp.int32, 16)
    w_splat = [plsc.load_gather(wb_v, [jnp.full((16,), 1 + d, jnp.int32)])
               for d in range(D)]
    b_splat = plsc.load_gather(wb_v, [jnp.full((16,), 1 + D, jnp.int32)])

    bufs = [(x_v0, sem0), (x_v1, sem1)]
    copies = [None] * NCH
    copies[0] = pltpu.async_copy(x_hbm.at[pl.ds(base, CH), :], x_v0, sem0)

    for k in range(NCH):
        x_v, _ = bufs[k % 2]
        copies[k].wait()
        if k + 1 < NCH:
            nxt, nsem = bufs[(k + 1) % 2]
            copies[k + 1] = pltpu.async_copy(
                x_hbm.at[pl.ds(base + (k + 1) * CH, CH), :], nxt, nsem)

        def block(blk, carry):
            rowids = blk * RB + lanes

            def step(l, acc):
                t = l + lanes
                lrot = jnp.where(t >= LSEQ, t - LSEQ, t)
                idx = plsc.load_gather(x_v, [rowids, lrot])
                idxd = idx * DP
                new = []
                for c in range(DP):
                    g = plsc.load_gather(emb_v, [idxd + c])
                    lo, hi = plsc.unpack(plsc.bitcast(g, jnp.int16),
                                         format=plsc.PackFormat.INTERLEAVED)
                    new.append(jnp.maximum(acc[2 * c], lo))
                    new.append(jnp.maximum(acc[2 * c + 1], hi))
                return tuple(new)

            init = tuple(jnp.full((16,), jnp.iinfo(jnp.int32).min, jnp.int32)
                         for _ in range(D))
            acc = lax.fori_loop(0, LSEQ, step, init)
            out16 = b_splat
            for d in range(D):
                out16 = out16 + w_splat[d] * acc[d].astype(jnp.float32)
            out_v[pl.ds(k * CH + blk * RB, RB)] = out16
            return carry

        lax.fori_loop(0, NBC, block, 0)

    pltpu.sync_copy(out_v, out_hbm.at[pl.ds(base, RPW)])


@jax.jit
def _set_nn(x, emb, W, b):
    mesh = plsc.VectorSubcoreMesh(core_axis_name="c", subcore_axis_name="s",
                                  num_cores=NC, num_subcores=NS)
    run = pl.kernel(
        _sc_body,
        out_type=jax.ShapeDtypeStruct((B,), jnp.float32),
        mesh=mesh,
        compiler_params=pltpu.CompilerParams(needs_layout_passes=False),
        scratch_types=[
            pltpu.VMEM((CH, LSEQ), jnp.int32),
            pltpu.VMEM((CH, LSEQ), jnp.int32),
            pltpu.VMEM((VOCAB * DP,), jnp.int32),
            pltpu.VMEM((16,), jnp.float32),
            pltpu.VMEM((RPW,), jnp.float32),
            pltpu.SemaphoreType.DMA,
            pltpu.SemaphoreType.DMA,
        ],
    )
    scale = 32766.0 / jnp.maximum(jnp.max(jnp.abs(emb)), 1e-30)
    q = jnp.round(emb * scale).astype(jnp.int16)
    wb = jnp.pad(jnp.concatenate([W.reshape(-1) / scale, b]), (1, 16 - D - 2))
    emb_packed = lax.bitcast_convert_type(
        q.reshape(VOCAB, DP, 2), jnp.int32).reshape(VOCAB * DP)
    out = run(x, emb_packed, wb)
    return out.reshape(B, 1)


def kernel(x, emb, W, b):
    return _set_nn(x, emb, W, b)


# speedup vs baseline: 1.3338x; 1.3338x over previous
import functools

import jax
import jax.numpy as jnp
from jax import lax
from jax.experimental import pallas as pl
from jax.experimental.pallas import tpu as pltpu
from jax.experimental.pallas import tpu_sc as plsc

B = 16384
LSEQ = 200
VOCAB = 1000
D = 10
DP = 5
NC, NS = 2, 16
NW = NC * NS
RPW = B // NW
RB = 16
CH = 128
NCH = RPW // CH
NBC = CH // RB


def _sc_body(x_hbm, emb_hbm, wb_hbm, out_hbm, x_v0, x_v1, emb_v, wb_v, out_v,
             sem0, sem1):
    wid = lax.axis_index("s") * NC + lax.axis_index("c")
    base = wid * RPW
    pltpu.sync_copy(emb_hbm, emb_v)
    pltpu.sync_copy(wb_hbm, wb_v)

    lanes = lax.iota(jnp.int32, 16)
    w_splat = [plsc.load_gather(wb_v, [jnp.full((16,), 1 + d, jnp.int32)])
               for d in range(D)]
    b_splat = plsc.load_gather(wb_v, [jnp.full((16,), 1 + D, jnp.int32)])

    bufs = [(x_v0, sem0), (x_v1, sem1)]
    copies = [None] * NCH
    copies[0] = pltpu.async_copy(x_hbm.at[pl.ds(base, CH), :], x_v0, sem0)

    for k in range(NCH):
        x_v, _ = bufs[k % 2]
        copies[k].wait()
        if k + 1 < NCH:
            nxt, nsem = bufs[(k + 1) % 2]
            copies[k + 1] = pltpu.async_copy(
                x_hbm.at[pl.ds(base + (k + 1) * CH, CH), :], nxt, nsem)

        def block(blk, carry):
            rowids = blk * RB + lanes

            def step(l, acc):
                t = l + lanes
                lrot = jnp.where(t >= LSEQ, t - LSEQ, t)
                idx = plsc.load_gather(x_v, [rowids, lrot])
                idxd = idx * DP
                return tuple(
                    jnp.maximum(acc[c],
                                plsc.bitcast(
                                    plsc.load_gather(emb_v, [idxd + c]),
                                    jnp.int16))
                    for c in range(DP))

            init = tuple(jnp.full((32,), -32768, jnp.int16)
                         for _ in range(DP))
            acc = lax.fori_loop(0, LSEQ, step, init)
            out16 = b_splat
            for c in range(DP):
                lo, hi = plsc.unpack(acc[c],
                                     format=plsc.PackFormat.INTERLEAVED)
                out16 = out16 + w_splat[2 * c] * lo.astype(jnp.float32)
                out16 = out16 + w_splat[2 * c + 1] * hi.astype(jnp.float32)
            out_v[pl.ds(k * CH + blk * RB, RB)] = out16
            return carry

        lax.fori_loop(0, NBC, block, 0)

    pltpu.sync_copy(out_v, out_hbm.at[pl.ds(base, RPW)])


@jax.jit
def _set_nn(x, emb, W, b):
    mesh = plsc.VectorSubcoreMesh(core_axis_name="c", subcore_axis_name="s",
                                  num_cores=NC, num_subcores=NS)
    run = pl.kernel(
        _sc_body,
        out_type=jax.ShapeDtypeStruct((B,), jnp.float32),
        mesh=mesh,
        compiler_params=pltpu.CompilerParams(needs_layout_passes=False),
        scratch_types=[
            pltpu.VMEM((CH, LSEQ), jnp.int32),
            pltpu.VMEM((CH, LSEQ), jnp.int32),
            pltpu.VMEM((VOCAB * DP,), jnp.int32),
            pltpu.VMEM((16,), jnp.float32),
            pltpu.VMEM((RPW,), jnp.float32),
            pltpu.SemaphoreType.DMA,
            pltpu.SemaphoreType.DMA,
        ],
    )
    scale = 32766.0 / jnp.maximum(jnp.max(jnp.abs(emb)), 1e-30)
    q = jnp.round(emb * scale).astype(jnp.int16)
    wb = jnp.pad(jnp.concatenate([W.reshape(-1) / scale, b]), (1, 16 - D - 2))
    emb_packed = lax.bitcast_convert_type(
        q.reshape(VOCAB, DP, 2), jnp.int32).reshape(VOCAB * DP)
    out = run(x, emb_packed, wb)
    return out.reshape(B, 1)


def kernel(x, emb, W, b):
    return _set_nn(x, emb, W, b)


# speedup vs baseline: 1.3371x; 1.0025x over previous
import functools

import jax
import jax.numpy as jnp
from jax import lax
from jax.experimental import pallas as pl
from jax.experimental.pallas import tpu as pltpu
from jax.experimental.pallas import tpu_sc as plsc

B = 16384
LSEQ = 200
VOCAB = 1000
D = 10
DP = 5
NC, NS = 2, 16
NW = NC * NS
RPW = B // NW
RB = 16
CH = 128
NCH = RPW // CH
NBC = CH // RB


def _sc_body(x_hbm, emb_hbm, wb_hbm, out_hbm, x_v0, x_v1, emb_v, wb_v, out_v,
             sem0, sem1):
    wid = lax.axis_index("s") * NC + lax.axis_index("c")
    base = wid * RPW
    pltpu.sync_copy(emb_hbm, emb_v)
    pltpu.sync_copy(wb_hbm, wb_v)

    lanes = lax.iota(jnp.int32, 16)
    w_splat = [plsc.load_gather(wb_v, [jnp.full((16,), 1 + d, jnp.int32)])
               for d in range(D)]
    b_splat = plsc.load_gather(wb_v, [jnp.full((16,), 1 + D, jnp.int32)])

    bufs = [(x_v0, sem0), (x_v1, sem1)]
    copies = [None] * NCH
    copies[0] = pltpu.async_copy(x_hbm.at[pl.ds(base, CH), :], x_v0, sem0)

    for k in range(NCH):
        x_v, _ = bufs[k % 2]
        copies[k].wait()
        if k + 1 < NCH:
            nxt, nsem = bufs[(k + 1) % 2]
            copies[k + 1] = pltpu.async_copy(
                x_hbm.at[pl.ds(base + (k + 1) * CH, CH), :], nxt, nsem)

        def block(blk, carry):
            rowids = blk * RB + lanes

            def step(l, acc):
                t = l + lanes
                lrot = jnp.where(t >= LSEQ, t - LSEQ, t)
                idx = plsc.load_gather(x_v, [rowids, lrot])
                idxd = idx * DP
                return tuple(
                    jnp.maximum(acc[c],
                                plsc.bitcast(
                                    plsc.load_gather(emb_v, [idxd + c]),
                                    jnp.int16))
                    for c in range(DP))

            init = tuple(jnp.full((32,), -32768, jnp.int16)
                         for _ in range(DP))
            acc = lax.fori_loop(0, LSEQ, step, init)
            out16 = b_splat
            for c in range(DP):
                lo, hi = plsc.unpack(acc[c],
                                     format=plsc.PackFormat.INTERLEAVED)
                out16 = out16 + w_splat[2 * c] * lo.astype(jnp.float32)
                out16 = out16 + w_splat[2 * c + 1] * hi.astype(jnp.float32)
            out_v[pl.ds(k * CH + blk * RB, RB)] = out16
            return carry

        lax.fori_loop(0, NBC, block, 0)

    pltpu.sync_copy(out_v, out_hbm.at[pl.ds(base, RPW)])


@jax.jit
def _set_nn(x, emb, W, b):
    mesh = plsc.VectorSubcoreMesh(core_axis_name="c", subcore_axis_name="s",
                                  num_cores=NC, num_subcores=NS)
    run = pl.kernel(
        _sc_body,
        out_type=jax.ShapeDtypeStruct((B,), jnp.float32),
        mesh=mesh,
        compiler_params=pltpu.CompilerParams(needs_layout_passes=False,
                                             use_tc_tiling_on_sc=True),
        scratch_types=[
            pltpu.VMEM((CH, LSEQ), jnp.int32),
            pltpu.VMEM((CH, LSEQ), jnp.int32),
            pltpu.VMEM((VOCAB * DP,), jnp.int32),
            pltpu.VMEM((16,), jnp.float32),
            pltpu.VMEM((RPW,), jnp.float32),
            pltpu.SemaphoreType.DMA,
            pltpu.SemaphoreType.DMA,
        ],
    )
    scale = 32766.0 / jnp.maximum(jnp.max(jnp.abs(emb)), 1e-30)
    q = jnp.round(emb * scale).astype(jnp.int16)
    wb = jnp.pad(jnp.concatenate([W.reshape(-1) / scale, b]), (1, 16 - D - 2))
    emb_packed = lax.bitcast_convert_type(
        q.reshape(VOCAB, DP, 2), jnp.int32).reshape(VOCAB * DP)
    out = run(x, emb_packed, wb)
    return out.reshape(B, 1)


def kernel(x, emb, W, b):
    return _set_nn(x, emb, W, b)


# speedup vs baseline: 1.3888x; 1.0387x over previous
import functools

import jax
import jax.numpy as jnp
from jax import lax
from jax.experimental import pallas as pl
from jax.experimental.pallas import tpu as pltpu
from jax.experimental.pallas import tpu_sc as plsc

B = 16384
LSEQ = 200
VOCAB = 1000
D = 10
DP = 5
NC, NS = 2, 16
NW = NC * NS
RPW = B // NW
RB = 16
CH = 128
NCH = RPW // CH
NBC = CH // RB


def _sc_body(x_hbm, emb_hbm, wb_hbm, out_hbm, x_v0, x_v1, emb_v, wb_v, out_v,
             sem0, sem1):
    wid = lax.axis_index("s") * NC + lax.axis_index("c")
    base = wid * RPW
    pltpu.sync_copy(emb_hbm, emb_v)
    pltpu.sync_copy(wb_hbm, wb_v)

    lanes = lax.iota(jnp.int32, 16)
    w_splat = [plsc.load_gather(wb_v, [jnp.full((16,), 1 + d, jnp.int32)])
               for d in range(D)]
    b_splat = plsc.load_gather(wb_v, [jnp.full((16,), 1 + D, jnp.int32)])

    bufs = [(x_v0, sem0), (x_v1, sem1)]
    copies = [None] * NCH
    copies[0] = pltpu.async_copy(x_hbm.at[pl.ds(base, CH), :], x_v0, sem0)

    for k in range(NCH):
        x_v, _ = bufs[k % 2]
        copies[k].wait()
        if k + 1 < NCH:
            nxt, nsem = bufs[(k + 1) % 2]
            copies[k + 1] = pltpu.async_copy(
                x_hbm.at[pl.ds(base + (k + 1) * CH, CH), :], nxt, nsem)

        def block(blk, carry):
            rowids = [blk * (2 * RB) + v * RB + lanes for v in range(2)]

            def step(l, acc):
                t = l + lanes
                lrot = jnp.where(t >= LSEQ, t - LSEQ, t)
                new = []
                for v in range(2):
                    idx = plsc.load_gather(x_v, [rowids[v], lrot])
                    idxd = idx * DP
                    new.extend(
                        jnp.maximum(acc[v * DP + c],
                                    plsc.bitcast(
                                        plsc.load_gather(emb_v, [idxd + c]),
                                        jnp.int16))
                        for c in range(DP))
                return tuple(new)

            init = tuple(jnp.full((32,), -32768, jnp.int16)
                         for _ in range(2 * DP))
            acc = lax.fori_loop(0, LSEQ, step, init)
            for v in range(2):
                out16 = b_splat
                for c in range(DP):
                    lo, hi = plsc.unpack(acc[v * DP + c],
                                         format=plsc.PackFormat.INTERLEAVED)
                    out16 = out16 + w_splat[2 * c] * lo.astype(jnp.float32)
                    out16 = out16 + w_splat[2 * c + 1] * hi.astype(jnp.float32)
                out_v[pl.ds(k * CH + blk * (2 * RB) + v * RB, RB)] = out16
            return carry

        lax.fori_loop(0, NBC // 2, block, 0)

    pltpu.sync_copy(out_v, out_hbm.at[pl.ds(base, RPW)])


@jax.jit
def _set_nn(x, emb, W, b):
    mesh = plsc.VectorSubcoreMesh(core_axis_name="c", subcore_axis_name="s",
                                  num_cores=NC, num_subcores=NS)
    run = pl.kernel(
        _sc_body,
        out_type=jax.ShapeDtypeStruct((B,), jnp.float32),
        mesh=mesh,
        compiler_params=pltpu.CompilerParams(needs_layout_passes=False,
                                             use_tc_tiling_on_sc=True),
        scratch_types=[
            pltpu.VMEM((CH, LSEQ), jnp.int32),
            pltpu.VMEM((CH, LSEQ), jnp.int32),
            pltpu.VMEM((VOCAB * DP,), jnp.int32),
            pltpu.VMEM((16,), jnp.float32),
            pltpu.VMEM((RPW,), jnp.float32),
            pltpu.SemaphoreType.DMA,
            pltpu.SemaphoreType.DMA,
        ],
    )
    scale = 32766.0 / jnp.maximum(jnp.max(jnp.abs(emb)), 1e-30)
    q = jnp.round(emb * scale).astype(jnp.int16)
    wb = jnp.pad(jnp.concatenate([W.reshape(-1) / scale, b]), (1, 16 - D - 2))
    emb_packed = lax.bitcast_convert_type(
        q.reshape(VOCAB, DP, 2), jnp.int32).reshape(VOCAB * DP)
    out = run(x, emb_packed, wb)
    return out.reshape(B, 1)


def kernel(x, emb, W, b):
    return _set_nn(x, emb, W, b)
